# baseline (speedup 1.0000x reference)
import functools

import jax
import jax.numpy as jnp
from jax import lax
from jax.experimental import pallas as pl
from jax.experimental.pallas import tpu as pltpu
from jax.experimental.pallas import tpu_sc as plsc

N_NODES = 100000
N_EDGES = 6400000

NC = 2
NS = 16
L = 16
NW = NC * NS
EPW = N_EDGES // NW
CH = 4000
NCH = EPW // CH
NB_T = 25
NPT = N_NODES // NB_T

_MESH = dict(core_axis_name="c", subcore_axis_name="s", num_cores=NC,
             num_subcores=NS)
_PARAMS = dict(
    mesh=plsc.VectorSubcoreMesh(**_MESH),
    compiler_params=pltpu.CompilerParams(needs_layout_passes=False),
)


def _wid():
    return lax.axis_index("s") * NC + lax.axis_index("c")


def _iota16():
    return lax.iota(jnp.int32, L)


def _rsqrt16(x):
    xi = plsc.bitcast(x, jnp.int32)
    yi = jnp.full((L,), 0x5F3759DF, jnp.int32) - lax.shift_right_logical(
        xi, jnp.full((L,), 1, jnp.int32))
    y = plsc.bitcast(yi, jnp.float32)
    half = jnp.full((L,), 0.5, jnp.float32)
    three_half = jnp.full((L,), 1.5, jnp.float32)
    for _ in range(3):
        y = y * (three_half - half * x * y * y)
    return y


def _deg_body(ei_hbm, ew_hbm, part_hbm, eiout_hbm, ebuf, wbuf, deg, dbuf):
    wid = _wid()
    zeros16 = jnp.zeros((L,), jnp.float32)

    def zinit(i, _):
        deg[pl.ds(i * L, L)] = zeros16
        return 0
    lax.fori_loop(0, N_NODES // L, zinit, 0)

    iota = _iota16()
    iota2 = iota * 2

    @pl.when(wid < NB_T)
    def _():
        tbase = wid * NPT

        def dinit(j, _):
            jidx = j * L + iota
            val = tbase + jidx
            pos = jidx * 2
            plsc.store_scatter(dbuf, [pos], val)
            plsc.store_scatter(dbuf, [pos + 1], val)
            return 0
        lax.fori_loop(0, NPT // L, dinit, 0)
        pltpu.sync_copy(
            dbuf, eiout_hbm.at[pl.ds((N_EDGES + tbase) * 2, NPT * 2)])

    def do_chunk(c, _):
        base = wid * EPW + c * CH
        pltpu.sync_copy(ei_hbm.at[pl.ds(base * 2, CH * 2)], ebuf)
        pltpu.sync_copy(ew_hbm.at[pl.ds(base, CH)], wbuf)
        pltpu.sync_copy(ebuf, eiout_hbm.at[pl.ds(base * 2, CH * 2)])

        def body(j, _):
            ridx = j * (2 * L) + iota2
            rows = plsc.load_gather(ebuf, [ridx])
            w = wbuf[pl.ds(j * L, L)]
            plsc.addupdate_scatter(deg, [rows], w)
            return 0
        lax.fori_loop(0, CH // L, body, 0)
        return 0
    lax.fori_loop(0, NCH, do_chunk, 0)

    pltpu.sync_copy(deg, part_hbm.at[pl.ds(wid * N_NODES, N_NODES)])


@jax.jit
def _deg_kernel(edge_index, edge_weight):
    return pl.kernel(
        _deg_body,
        out_type=(
            jax.ShapeDtypeStruct((NW * N_NODES,), jnp.float32),
            jax.ShapeDtypeStruct(((N_EDGES + N_NODES) * 2,), jnp.int32),
        ),
        scratch_types=[
            pltpu.VMEM((CH * 2,), jnp.int32),
            pltpu.VMEM((CH,), jnp.float32),
            pltpu.VMEM((N_NODES,), jnp.float32),
            pltpu.VMEM((NPT * 2,), jnp.int32),
        ],
        **_PARAMS,
    )(edge_index.reshape(N_EDGES * 2), edge_weight)


def _reduce_body(part_hbm, dis_hbm, acc, buf, disb):
    wid = _wid()

    @pl.when(wid < NB_T)
    def _():
        base = wid * NPT
        ones16 = jnp.full((L,), 1.0, jnp.float32)

        def init(i, _):
            acc[pl.ds(i * L, L)] = ones16
            return 0
        lax.fori_loop(0, NPT // L, init, 0)

        for k in range(NW):
            pltpu.sync_copy(part_hbm.at[pl.ds(k * N_NODES + base, NPT)], buf)

            def add(i, _):
                s = pl.ds(i * L, L)
                acc[s] = acc[s] + buf[s]
                return 0
            lax.fori_loop(0, NPT // L, add, 0)

        def finish(i, _):
            s = pl.ds(i * L, L)
            disb[s] = _rsqrt16(acc[s])
            return 0
        lax.fori_loop(0, NPT // L, finish, 0)

        pltpu.sync_copy(disb, dis_hbm.at[pl.ds(base, NPT)])


@jax.jit
def _reduce_kernel(part):
    return pl.kernel(
        _reduce_body,
        out_type=jax.ShapeDtypeStruct((N_NODES,), jnp.float32),
        scratch_types=[
            pltpu.VMEM((NPT,), jnp.float32),
            pltpu.VMEM((NPT,), jnp.float32),
            pltpu.VMEM((NPT,), jnp.float32),
        ],
        **_PARAMS,
    )(part)


def _norm_body(ei_hbm, ew_hbm, dis_hbm, out_hbm, disb, ebuf, wbuf, obuf):
    wid = _wid()
    pltpu.sync_copy(dis_hbm, disb)

    iota2 = _iota16() * 2
    ones16i = jnp.ones((L,), jnp.int32)

    @pl.when(wid < NB_T)
    def _():
        def sbody(i, _):
            v = disb[pl.ds(wid * NPT + i * L, L)]
            obuf[pl.ds(i * L, L)] = v * v
            return 0
        lax.fori_loop(0, NPT // L, sbody, 0)
        pltpu.sync_copy(obuf, out_hbm.at[pl.ds(N_EDGES + wid * NPT, NPT)])

    def do_chunk(c, _):
        base = wid * EPW + c * CH
        pltpu.sync_copy(ei_hbm.at[pl.ds(base * 2, CH * 2)], ebuf)
        pltpu.sync_copy(ew_hbm.at[pl.ds(base, CH)], wbuf)

        def body(j, _):
            ridx = j * (2 * L) + iota2
            rows = plsc.load_gather(ebuf, [ridx])
            cols = plsc.load_gather(ebuf, [ridx + ones16i])
            dr = plsc.load_gather(disb, [rows])
            dc = plsc.load_gather(disb, [cols])
            w = wbuf[pl.ds(j * L, L)]
            obuf[pl.ds(j * L, L)] = dr * w * dc
            return 0
        lax.fori_loop(0, CH // L, body, 0)

        pltpu.sync_copy(obuf, out_hbm.at[pl.ds(base, CH)])
        return 0
    lax.fori_loop(0, NCH, do_chunk, 0)


@jax.jit
def _norm_kernel(edge_index, edge_weight, dis):
    return pl.kernel(
        _norm_body,
        out_type=jax.ShapeDtypeStruct((N_EDGES + N_NODES,), jnp.float32),
        scratch_types=[
            pltpu.VMEM((N_NODES,), jnp.float32),
            pltpu.VMEM((CH * 2,), jnp.int32),
            pltpu.VMEM((CH,), jnp.float32),
            pltpu.VMEM((CH,), jnp.float32),
        ],
        **_PARAMS,
    )(edge_index.reshape(N_EDGES * 2), edge_weight, dis)


def kernel(edge_index, edge_weight):
    part, ei_flat = _deg_kernel(edge_index, edge_weight)
    dis = _reduce_kernel(part)
    normed = _norm_kernel(edge_index, edge_weight, dis)
    return (ei_flat.reshape(N_EDGES + N_NODES, 2), normed)


# speedup vs baseline: 23.2604x; 23.2604x over previous
import functools

import jax
import jax.numpy as jnp
from jax import lax
from jax.experimental import pallas as pl
from jax.experimental.pallas import tpu as pltpu
from jax.experimental.pallas import tpu_sc as plsc

N_NODES = 100000
N_EDGES = 6400000

NC = 2
NS = 16
L = 16
NW = NC * NS
EPW = N_EDGES // NW
CH = 4000
NCH = EPW // CH
NB_T = 25
NPT = N_NODES // NB_T

_MESH = dict(core_axis_name="c", subcore_axis_name="s", num_cores=NC,
             num_subcores=NS)
_PARAMS = dict(
    mesh=plsc.VectorSubcoreMesh(**_MESH),
    compiler_params=pltpu.CompilerParams(needs_layout_passes=False),
)


def _wid():
    return lax.axis_index("s") * NC + lax.axis_index("c")


def _iota16():
    return lax.iota(jnp.int32, L)


def _rsqrt16(x):
    xi = plsc.bitcast(x, jnp.int32)
    yi = jnp.full((L,), 0x5F3759DF, jnp.int32) - lax.shift_right_logical(
        xi, jnp.full((L,), 1, jnp.int32))
    y = plsc.bitcast(yi, jnp.float32)
    half = jnp.full((L,), 0.5, jnp.float32)
    three_half = jnp.full((L,), 1.5, jnp.float32)
    for _ in range(3):
        y = y * (three_half - half * x * y * y)
    return y


def _deg_body(row_hbm, ew_hbm, part_hbm, rbuf, wbuf, deg):
    wid = _wid()
    zeros16 = jnp.zeros((L,), jnp.float32)

    def zinit(i, _):
        deg[pl.ds(i * L, L)] = zeros16
        return 0
    lax.fori_loop(0, N_NODES // L, zinit, 0)

    def do_chunk(c, _):
        base = wid * EPW + c * CH
        pltpu.sync_copy(row_hbm.at[pl.ds(base, CH)], rbuf)
        pltpu.sync_copy(ew_hbm.at[pl.ds(base, CH)], wbuf)

        def body(j, _):
            s = pl.ds(j * L, L)
            plsc.addupdate_scatter(deg, [rbuf[s]], wbuf[s])
            return 0
        lax.fori_loop(0, CH // L, body, 0)
        return 0
    lax.fori_loop(0, NCH, do_chunk, 0)

    pltpu.sync_copy(deg, part_hbm.at[pl.ds(wid * N_NODES, N_NODES)])


@jax.jit
def _deg_kernel(row, edge_weight):
    return pl.kernel(
        _deg_body,
        out_type=jax.ShapeDtypeStruct((NW * N_NODES,), jnp.float32),
        scratch_types=[
            pltpu.VMEM((CH,), jnp.int32),
            pltpu.VMEM((CH,), jnp.float32),
            pltpu.VMEM((N_NODES,), jnp.float32),
        ],
        **_PARAMS,
    )(row, edge_weight)


def _reduce_body(part_hbm, dis_hbm, acc, buf, disb):
    wid = _wid()

    @pl.when(wid < NB_T)
    def _():
        base = wid * NPT
        ones16 = jnp.full((L,), 1.0, jnp.float32)

        def init(i, _):
            acc[pl.ds(i * L, L)] = ones16
            return 0
        lax.fori_loop(0, NPT // L, init, 0)

        for k in range(NW):
            pltpu.sync_copy(part_hbm.at[pl.ds(k * N_NODES + base, NPT)], buf)

            def add(i, _):
                s = pl.ds(i * L, L)
                acc[s] = acc[s] + buf[s]
                return 0
            lax.fori_loop(0, NPT // L, add, 0)

        def finish(i, _):
            s = pl.ds(i * L, L)
            disb[s] = _rsqrt16(acc[s])
            return 0
        lax.fori_loop(0, NPT // L, finish, 0)

        pltpu.sync_copy(disb, dis_hbm.at[pl.ds(base, NPT)])


@jax.jit
def _reduce_kernel(part):
    return pl.kernel(
        _reduce_body,
        out_type=jax.ShapeDtypeStruct((N_NODES,), jnp.float32),
        scratch_types=[
            pltpu.VMEM((NPT,), jnp.float32),
            pltpu.VMEM((NPT,), jnp.float32),
            pltpu.VMEM((NPT,), jnp.float32),
        ],
        **_PARAMS,
    )(part)


def _norm_body(row_hbm, col_hbm, ew_hbm, dis_hbm, out_hbm,
               disb, rbuf, cbuf, wbuf, obuf):
    wid = _wid()
    pltpu.sync_copy(dis_hbm, disb)

    @pl.when(wid < NB_T)
    def _():
        def sbody(i, _):
            v = disb[pl.ds(wid * NPT + i * L, L)]
            obuf[pl.ds(i * L, L)] = v * v
            return 0
        lax.fori_loop(0, NPT // L, sbody, 0)
        pltpu.sync_copy(obuf, out_hbm.at[pl.ds(N_EDGES + wid * NPT, NPT)])

    def do_chunk(c, _):
        base = wid * EPW + c * CH
        pltpu.sync_copy(row_hbm.at[pl.ds(base, CH)], rbuf)
        pltpu.sync_copy(col_hbm.at[pl.ds(base, CH)], cbuf)
        pltpu.sync_copy(ew_hbm.at[pl.ds(base, CH)], wbuf)

        def body(j, _):
            s = pl.ds(j * L, L)
            dr = plsc.load_gather(disb, [rbuf[s]])
            dc = plsc.load_gather(disb, [cbuf[s]])
            obuf[s] = dr * wbuf[s] * dc
            return 0
        lax.fori_loop(0, CH // L, body, 0)

        pltpu.sync_copy(obuf, out_hbm.at[pl.ds(base, CH)])
        return 0
    lax.fori_loop(0, NCH, do_chunk, 0)


@jax.jit
def _norm_kernel(row, col, edge_weight, dis):
    return pl.kernel(
        _norm_body,
        out_type=jax.ShapeDtypeStruct((N_EDGES + N_NODES,), jnp.float32),
        scratch_types=[
            pltpu.VMEM((N_NODES,), jnp.float32),
            pltpu.VMEM((CH,), jnp.int32),
            pltpu.VMEM((CH,), jnp.int32),
            pltpu.VMEM((CH,), jnp.float32),
            pltpu.VMEM((CH,), jnp.float32),
        ],
        **_PARAMS,
    )(row, col, edge_weight, dis)


def kernel(edge_index, edge_weight):
    row = edge_index[:, 0]
    col = edge_index[:, 1]
    diag = jnp.arange(N_NODES, dtype=edge_index.dtype)
    ei = jnp.concatenate(
        [edge_index, jnp.stack([diag, diag], axis=1)], axis=0)
    part = _deg_kernel(row, edge_weight)
    dis = _reduce_kernel(part)
    normed = _norm_kernel(row, col, edge_weight, dis)
    return (ei, normed)


# speedup vs baseline: 36.6384x; 1.5751x over previous
import jax
import jax.numpy as jnp
from jax import lax
from jax.experimental import pallas as pl
from jax.experimental.pallas import tpu as pltpu
from jax.experimental.pallas import tpu_sc as plsc

N_NODES = 100000
N_EDGES = 6400000

NC = 2
NS = 16
L = 16
NW = NC * NS
EPW = N_EDGES // NW
CHA = 4000
NCHA = EPW // CHA
CHC = 2000
NCHC = EPW // CHC
NB_T = 25
NPT = N_NODES // NB_T

_MESH = dict(core_axis_name="c", subcore_axis_name="s", num_cores=NC,
             num_subcores=NS)
_PARAMS = dict(
    mesh=plsc.VectorSubcoreMesh(**_MESH),
    compiler_params=pltpu.CompilerParams(needs_layout_passes=False),
)


def _wid():
    return lax.axis_index("s") * NC + lax.axis_index("c")


def _rsqrt16(x):
    xi = plsc.bitcast(x, jnp.int32)
    yi = jnp.full((L,), 0x5F3759DF, jnp.int32) - lax.shift_right_logical(
        xi, jnp.full((L,), 1, jnp.int32))
    y = plsc.bitcast(yi, jnp.float32)
    half = jnp.full((L,), 0.5, jnp.float32)
    three_half = jnp.full((L,), 1.5, jnp.float32)
    for _ in range(3):
        y = y * (three_half - half * x * y * y)
    return y


def _deg_body(row_hbm, ew_hbm, part_hbm,
              rb0, rb1, wb0, wb1, deg, sr0, sr1, sw0, sw1):
    wid = _wid()
    ebase = wid * EPW
    bufs = ((rb0, wb0, sr0, sw0), (rb1, wb1, sr1, sw1))
    zeros16 = jnp.zeros((L,), jnp.float32)

    def zinit(i, _):
        deg[pl.ds(i * L, L)] = zeros16
        return 0
    lax.fori_loop(0, N_NODES // L, zinit, 0, unroll=8)

    def issue(b, c):
        base = ebase + c * CHA
        rb, wb, sr, sw = bufs[b]
        pltpu.async_copy(row_hbm.at[pl.ds(base, CHA)], rb, sr)
        pltpu.async_copy(ew_hbm.at[pl.ds(base, CHA)], wb, sw)

    issue(0, 0)
    issue(1, 1)

    def outer(g, _):
        for b in range(2):
            cg = g * 2 + b
            rb, wb, sr, sw = bufs[b]
            pltpu.make_async_copy(row_hbm.at[pl.ds(0, CHA)], rb, sr).wait()
            pltpu.make_async_copy(ew_hbm.at[pl.ds(0, CHA)], wb, sw).wait()

            def body(j, _):
                s = pl.ds(j * L, L)
                plsc.addupdate_scatter(deg, [rb[s]], wb[s])
                return 0
            lax.fori_loop(0, CHA // L, body, 0, unroll=4)

            @pl.when(cg + 2 < NCHA)
            def _():
                issue(b, cg + 2)
        return 0
    lax.fori_loop(0, NCHA // 2, outer, 0)

    pltpu.sync_copy(deg, part_hbm.at[pl.ds(wid * N_NODES, N_NODES)])


@jax.jit
def _deg_kernel(row, edge_weight):
    return pl.kernel(
        _deg_body,
        out_type=jax.ShapeDtypeStruct((NW * N_NODES,), jnp.float32),
        scratch_types=[
            pltpu.VMEM((CHA,), jnp.int32),
            pltpu.VMEM((CHA,), jnp.int32),
            pltpu.VMEM((CHA,), jnp.float32),
            pltpu.VMEM((CHA,), jnp.float32),
            pltpu.VMEM((N_NODES,), jnp.float32),
            pltpu.SemaphoreType.DMA,
            pltpu.SemaphoreType.DMA,
            pltpu.SemaphoreType.DMA,
            pltpu.SemaphoreType.DMA,
        ],
        **_PARAMS,
    )(row, edge_weight)


_NRING = 4


def _reduce_body(part_hbm, dis_hbm, acc, b0, b1, b2, b3, s0, s1, s2, s3):
    wid = _wid()
    bufs = ((b0, s0), (b1, s1), (b2, s2), (b3, s3))

    @pl.when(wid < NB_T)
    def _():
        base = wid * NPT
        ones16 = jnp.full((L,), 1.0, jnp.float32)

        def issue(r, k):
            buf, sem = bufs[r]
            pltpu.async_copy(
                part_hbm.at[pl.ds(k * N_NODES + base, NPT)], buf, sem)

        for r in range(_NRING):
            issue(r, r)

        def init(i, _):
            acc[pl.ds(i * L, L)] = ones16
            return 0
        lax.fori_loop(0, NPT // L, init, 0, unroll=8)

        def outer(g, _):
            for r in range(_NRING):
                k = g * _NRING + r
                buf, sem = bufs[r]
                pltpu.make_async_copy(
                    part_hbm.at[pl.ds(0, NPT)], buf, sem).wait()

                def add(i, _):
                    s = pl.ds(i * L, L)
                    acc[s] = acc[s] + buf[s]
                    return 0
                lax.fori_loop(0, NPT // L, add, 0, unroll=8)

                @pl.when(k + _NRING < NW)
                def _():
                    issue(r, k + _NRING)
            return 0
        lax.fori_loop(0, NW // _NRING, outer, 0)

        def finish(i, _):
            s = pl.ds(i * L, L)
            acc[s] = _rsqrt16(acc[s])
            return 0
        lax.fori_loop(0, NPT // L, finish, 0, unroll=4)

        pltpu.sync_copy(acc, dis_hbm.at[pl.ds(base, NPT)])


@jax.jit
def _reduce_kernel(part):
    return pl.kernel(
        _reduce_body,
        out_type=jax.ShapeDtypeStruct((N_NODES,), jnp.float32),
        scratch_types=[
            pltpu.VMEM((NPT,), jnp.float32),
            pltpu.VMEM((NPT,), jnp.float32),
            pltpu.VMEM((NPT,), jnp.float32),
            pltpu.VMEM((NPT,), jnp.float32),
            pltpu.VMEM((NPT,), jnp.float32),
            pltpu.SemaphoreType.DMA,
            pltpu.SemaphoreType.DMA,
            pltpu.SemaphoreType.DMA,
            pltpu.SemaphoreType.DMA,
        ],
        **_PARAMS,
    )(part)


def _norm_body(row_hbm, col_hbm, ew_hbm, dis_hbm, out_hbm, disb, tbuf,
               rb0, rb1, cb0, cb1, wb0, wb1, ob0, ob1,
               sd, st, sr0, sr1, sc0, sc1, sw0, sw1, so0, so1):
    wid = _wid()
    ebase = wid * EPW
    bufs = ((rb0, cb0, wb0, ob0, sr0, sc0, sw0, so0),
            (rb1, cb1, wb1, ob1, sr1, sc1, sw1, so1))

    cpdis = pltpu.async_copy(dis_hbm, disb, sd)

    def issue(b, c):
        base = ebase + c * CHC
        rb, cb, wb = bufs[b][0], bufs[b][1], bufs[b][2]
        sr, sc, sw = bufs[b][4], bufs[b][5], bufs[b][6]
        pltpu.async_copy(row_hbm.at[pl.ds(base, CHC)], rb, sr)
        pltpu.async_copy(col_hbm.at[pl.ds(base, CHC)], cb, sc)
        pltpu.async_copy(ew_hbm.at[pl.ds(base, CHC)], wb, sw)

    issue(0, 0)
    issue(1, 1)
    cpdis.wait()

    @pl.when(wid < NB_T)
    def _():
        def sbody(i, _):
            v = disb[pl.ds(wid * NPT + i * L, L)]
            tbuf[pl.ds(i * L, L)] = v * v
            return 0
        lax.fori_loop(0, NPT // L, sbody, 0, unroll=4)
        pltpu.async_copy(
            tbuf, out_hbm.at[pl.ds(N_EDGES + wid * NPT, NPT)], st)

    def outer(g, _):
        for b in range(2):
            cg = g * 2 + b
            rb, cb, wb, ob, sr, sc, sw, so = bufs[b]
            pltpu.make_async_copy(row_hbm.at[pl.ds(0, CHC)], rb, sr).wait()
            pltpu.make_async_copy(col_hbm.at[pl.ds(0, CHC)], cb, sc).wait()
            pltpu.make_async_copy(ew_hbm.at[pl.ds(0, CHC)], wb, sw).wait()

            @pl.when(cg >= 2)
            def _():
                pltpu.make_async_copy(
                    ob, out_hbm.at[pl.ds(0, CHC)], so).wait()

            def body(j, _):
                s = pl.ds(j * L, L)
                dr = plsc.load_gather(disb, [rb[s]])
                dc = plsc.load_gather(disb, [cb[s]])
                ob[s] = dr * wb[s] * dc
                return 0
            lax.fori_loop(0, CHC // L, body, 0, unroll=4)

            pltpu.async_copy(ob, out_hbm.at[pl.ds(ebase + cg * CHC, CHC)], so)

            @pl.when(cg + 2 < NCHC)
            def _():
                issue(b, cg + 2)
        return 0
    lax.fori_loop(0, NCHC // 2, outer, 0)

    for b in range(2):
        ob, so = bufs[b][3], bufs[b][7]
        pltpu.make_async_copy(ob, out_hbm.at[pl.ds(0, CHC)], so).wait()

    @pl.when(wid < NB_T)
    def _():
        pltpu.make_async_copy(
            tbuf, out_hbm.at[pl.ds(0, NPT)], st).wait()


@jax.jit
def _norm_kernel(row, col, edge_weight, dis):
    return pl.kernel(
        _norm_body,
        out_type=jax.ShapeDtypeStruct((N_EDGES + N_NODES,), jnp.float32),
        scratch_types=[
            pltpu.VMEM((N_NODES,), jnp.float32),
            pltpu.VMEM((NPT,), jnp.float32),
            pltpu.VMEM((CHC,), jnp.int32),
            pltpu.VMEM((CHC,), jnp.int32),
            pltpu.VMEM((CHC,), jnp.int32),
            pltpu.VMEM((CHC,), jnp.int32),
            pltpu.VMEM((CHC,), jnp.float32),
            pltpu.VMEM((CHC,), jnp.float32),
            pltpu.VMEM((CHC,), jnp.float32),
            pltpu.VMEM((CHC,), jnp.float32),
            pltpu.SemaphoreType.DMA,
            pltpu.SemaphoreType.DMA,
            pltpu.SemaphoreType.DMA,
            pltpu.SemaphoreType.DMA,
            pltpu.SemaphoreType.DMA,
            pltpu.SemaphoreType.DMA,
            pltpu.SemaphoreType.DMA,
            pltpu.SemaphoreType.DMA,
            pltpu.SemaphoreType.DMA,
            pltpu.SemaphoreType.DMA,
        ],
        **_PARAMS,
    )(row, col, edge_weight, dis)


def kernel(edge_index, edge_weight):
    row = edge_index[:, 0]
    col = edge_index[:, 1]
    diag = jnp.arange(N_NODES, dtype=edge_index.dtype)
    ei = jnp.concatenate(
        [edge_index, jnp.stack([diag, diag], axis=1)], axis=0)
    part = _deg_kernel(row, edge_weight)
    dis = _reduce_kernel(part)
    normed = _norm_kernel(row, col, edge_weight, dis)
    return (ei, normed)
